# Initial kernel scaffold; baseline (speedup 1.0000x reference)
import functools

import jax
import jax.numpy as jnp
from jax import lax
from jax.experimental import pallas as pl
from jax.experimental.pallas import tpu as pltpu
from jax.experimental.pallas import tpu_sc as plsc

_N = 10000
_E = 320000
_D = 128
_H1 = 128
_H2 = 64
_G = 64
_C = 10

_NC = 2
_NS = 16
_NW = _NC * _NS
_CHUNK = 128
_CPW = -(-_E // (_NW * _CHUNK))
_EPAD = _NW * _CPW * _CHUNK
_DW = 16
_N2 = _N + 16
_RPT = _N2 // _NS

_RB = 2000


def _sc_mesh():
    return plsc.VectorSubcoreMesh(
        core_axis_name="c", subcore_axis_name="s",
        num_cores=_NC, num_subcores=_NS)


def _make_deg_kernel():
    @functools.partial(
        pl.kernel,
        mesh=_sc_mesh(),
        out_type=jax.ShapeDtypeStruct((_NC, _N2, _DW), jnp.float32),
        scratch_types=[
            pltpu.VMEM((_CPW, _CHUNK), jnp.int32),
            pltpu.VMEM((_CHUNK, _DW), jnp.float32),
            pltpu.VMEM_SHARED((_N2, _DW), jnp.float32),
            pltpu.SemaphoreType.DMA,
        ],
    )
    def deg_kernel(dst_hbm, ones_hbm, zeros_hbm, out_hbm, dst_v, ones_v, acc, sem):
        cid = lax.axis_index("c")
        sid = lax.axis_index("s")
        wid = sid * _NC + cid
        pltpu.sync_copy(dst_hbm.at[wid], dst_v)
        pltpu.sync_copy(ones_hbm, ones_v)
        pltpu.sync_copy(zeros_hbm.at[pl.ds(sid * _RPT, _RPT)],
                        acc.at[pl.ds(sid * _RPT, _RPT)])
        plsc.subcore_barrier()

        def body(j, carry):
            pltpu.sync_copy(ones_v, acc.at[dst_v.at[j]], add=True)
            return carry

        lax.fori_loop(0, _CPW, body, 0)
        plsc.subcore_barrier()
        pltpu.sync_copy(acc.at[pl.ds(sid * _RPT, _RPT)],
                        out_hbm.at[cid, pl.ds(sid * _RPT, _RPT)])

    return deg_kernel


def _make_spmm_kernel(d):
    @functools.partial(
        pl.kernel,
        mesh=_sc_mesh(),
        out_type=jax.ShapeDtypeStruct((_NC, _N2, d), jnp.float32),
        scratch_types=[
            pltpu.VMEM((_CPW, _CHUNK), jnp.int32),
            pltpu.VMEM((_CPW, _CHUNK), jnp.int32),
            pltpu.VMEM((_CHUNK, d), jnp.float32),
            pltpu.VMEM_SHARED((_N2, d), jnp.float32),
            pltpu.SemaphoreType.DMA,
        ],
    )
    def spmm_kernel(g_hbm, src_hbm, dst_hbm, zeros_hbm, out_hbm,
                    src_v, dst_v, rows_v, acc, sem):
        cid = lax.axis_index("c")
        sid = lax.axis_index("s")
        wid = sid * _NC + cid
        pltpu.sync_copy(src_hbm.at[wid], src_v)
        pltpu.sync_copy(dst_hbm.at[wid], dst_v)
        pltpu.sync_copy(zeros_hbm.at[pl.ds(sid * _RPT, _RPT)],
                        acc.at[pl.ds(sid * _RPT, _RPT)])
        plsc.subcore_barrier()

        def body(j, carry):
            pltpu.async_copy(g_hbm.at[src_v.at[j]], rows_v, sem).wait()
            pltpu.sync_copy(rows_v, acc.at[dst_v.at[j]], add=True)
            return carry

        lax.fori_loop(0, _CPW, body, 0)
        plsc.subcore_barrier()
        pltpu.sync_copy(acc.at[pl.ds(sid * _RPT, _RPT)],
                        out_hbm.at[cid, pl.ds(sid * _RPT, _RPT)])

    return spmm_kernel


_deg_kernel = _make_deg_kernel()
_spmm128 = _make_spmm_kernel(_H1)
_spmm64 = _make_spmm_kernel(_H2)


def _g1_body(degp_ref, x_ref, w1_ref, g1_ref, dinv_ref):
    deg = degp_ref[0, :, 0:1] + degp_ref[1, :, 0:1] + 1.0
    dinv = lax.rsqrt(deg)
    h = jnp.dot(x_ref[...], w1_ref[...], preferred_element_type=jnp.float32)
    g1_ref[...] = h * dinv
    dinv_ref[...] = dinv


def _layer2_body(s1p_ref, g1_ref, dinv_ref, b1_ref, w2_ref, g2_ref):
    s = s1p_ref[0] + s1p_ref[1] + g1_ref[...]
    h1 = jnp.maximum(s * dinv_ref[...] + b1_ref[...], 0.0)
    g2_ref[...] = jnp.dot(h1, w2_ref[...],
                          preferred_element_type=jnp.float32) * dinv_ref[...]


def _pool_body(s2p_ref, g2_ref, dinv_ref, b2_ref, batch_ref, wfc_ref, bfc_ref,
               out_ref, pooled, counts):
    i = pl.program_id(0)

    @pl.when(i == 0)
    def _init():
        pooled[...] = jnp.zeros_like(pooled)
        counts[...] = jnp.zeros_like(counts)

    h2 = (s2p_ref[0] + s2p_ref[1] + g2_ref[...]) * dinv_ref[...] + b2_ref[...]
    gid = lax.broadcasted_iota(jnp.int32, (_G, _RB), 0)
    mask = jnp.where(gid == batch_ref[...], 1.0, 0.0)
    pooled[...] += jnp.dot(mask, h2, preferred_element_type=jnp.float32)
    counts[...] += jnp.sum(mask, axis=1, keepdims=True)

    @pl.when(i == pl.num_programs(0) - 1)
    def _fin():
        p = pooled[...] / jnp.maximum(counts[...], 1.0)
        out_ref[...] = jnp.dot(p, wfc_ref[...],
                               preferred_element_type=jnp.float32) + bfc_ref[...]


def kernel(x, edge_index, batch, W1, b1, W2, b2, Wfc, bfc):
    grid = _N // _RB

    src = edge_index[0]
    dst = edge_index[1]
    pad = jnp.arange(_EPAD - _E, dtype=jnp.int32) % 16
    src3 = jnp.concatenate([src, pad]).reshape(_NW, _CPW, _CHUNK)
    dst3 = jnp.concatenate([dst, _N + pad]).reshape(_NW, _CPW, _CHUNK)

    ones_dw = jnp.ones((_CHUNK, _DW), jnp.float32)
    zeros_dw = jnp.zeros((_N2, _DW), jnp.float32)
    zeros_h1 = jnp.zeros((_N2, _H1), jnp.float32)
    zeros_h2 = jnp.zeros((_N2, _H2), jnp.float32)

    degp = _deg_kernel(dst3, ones_dw, zeros_dw)

    g1, dinv = pl.pallas_call(
        _g1_body,
        grid=(grid,),
        in_specs=[
            pl.BlockSpec((_NC, _RB, _DW), lambda i: (0, i, 0)),
            pl.BlockSpec((_RB, _D), lambda i: (i, 0)),
            pl.BlockSpec((_D, _H1), lambda i: (0, 0)),
        ],
        out_specs=[
            pl.BlockSpec((_RB, _H1), lambda i: (i, 0)),
            pl.BlockSpec((_RB, 1), lambda i: (i, 0)),
        ],
        out_shape=[
            jax.ShapeDtypeStruct((_N, _H1), jnp.float32),
            jax.ShapeDtypeStruct((_N, 1), jnp.float32),
        ],
    )(degp, x, W1)

    s1p = _spmm128(g1, src3, dst3, zeros_h1)

    g2 = pl.pallas_call(
        _layer2_body,
        grid=(grid,),
        in_specs=[
            pl.BlockSpec((_NC, _RB, _H1), lambda i: (0, i, 0)),
            pl.BlockSpec((_RB, _H1), lambda i: (i, 0)),
            pl.BlockSpec((_RB, 1), lambda i: (i, 0)),
            pl.BlockSpec((1, _H1), lambda i: (0, 0)),
            pl.BlockSpec((_H1, _H2), lambda i: (0, 0)),
        ],
        out_specs=pl.BlockSpec((_RB, _H2), lambda i: (i, 0)),
        out_shape=jax.ShapeDtypeStruct((_N, _H2), jnp.float32),
    )(s1p, g1, dinv, b1.reshape(1, _H1), W2)

    s2p = _spmm64(g2, src3, dst3, zeros_h2)

    out = pl.pallas_call(
        _pool_body,
        grid=(grid,),
        in_specs=[
            pl.BlockSpec((_NC, _RB, _H2), lambda i: (0, i, 0)),
            pl.BlockSpec((_RB, _H2), lambda i: (i, 0)),
            pl.BlockSpec((_RB, 1), lambda i: (i, 0)),
            pl.BlockSpec((1, _H2), lambda i: (0, 0)),
            pl.BlockSpec((1, _RB), lambda i: (0, i)),
            pl.BlockSpec((_H2, _C), lambda i: (0, 0)),
            pl.BlockSpec((1, _C), lambda i: (0, 0)),
        ],
        out_specs=pl.BlockSpec((_G, _C), lambda i: (0, 0)),
        out_shape=jax.ShapeDtypeStruct((_G, _C), jnp.float32),
        scratch_shapes=[
            pltpu.VMEM((_G, _H2), jnp.float32),
            pltpu.VMEM((_G, 1), jnp.float32),
        ],
    )(s2p, g2, dinv, b2.reshape(1, _N), batch.reshape(1, _N), Wfc,
      bfc.reshape(1, _C))

    return out


# speedup vs baseline: 24.8738x; 24.8738x over previous
import functools

import jax
import jax.numpy as jnp
from jax import lax
from jax.experimental import pallas as pl
from jax.experimental.pallas import tpu as pltpu
from jax.experimental.pallas import tpu_sc as plsc

_N = 10000
_E = 320000
_D = 128
_H1 = 128
_H2 = 64
_G = 64
_C = 10

_NC = 2
_NS = 16
_NW = _NC * _NS
_CHUNK = 128
_CPW = -(-_E // (_NW * _CHUNK))
_EPAD = _NW * _CPW * _CHUNK
_DW = 16
_N2 = 10112
_RPT = _N2 // _NS

_RB = 2000


def _sc_mesh():
    return plsc.VectorSubcoreMesh(
        core_axis_name="c", subcore_axis_name="s",
        num_cores=_NC, num_subcores=_NS)


def _make_deg_kernel():
    @functools.partial(
        pl.kernel,
        mesh=_sc_mesh(),
        compiler_params=pltpu.CompilerParams(use_tc_tiling_on_sc=False),
        out_type=jax.ShapeDtypeStruct((_NC, _N2, _DW), jnp.float32),
        scratch_types=[
            pltpu.VMEM((_CPW, _CHUNK), jnp.int32),
            pltpu.VMEM((_CHUNK, _DW), jnp.float32),
            pltpu.VMEM_SHARED((_N2, _DW), jnp.float32),
            pltpu.SemaphoreType.DMA,
        ],
    )
    def deg_kernel(dst_hbm, ones_hbm, zeros_hbm, out_hbm, dst_v, ones_v, acc, sem):
        cid = lax.axis_index("c")
        sid = lax.axis_index("s")
        wid = sid * _NC + cid
        pltpu.sync_copy(dst_hbm.at[wid], dst_v)
        pltpu.sync_copy(ones_hbm, ones_v)
        pltpu.sync_copy(zeros_hbm.at[pl.ds(sid * _RPT, _RPT)],
                        acc.at[pl.ds(sid * _RPT, _RPT)])
        plsc.subcore_barrier()

        def body(j, carry):
            pltpu.sync_copy(ones_v, acc.at[dst_v.at[j]], add=True)
            return carry

        lax.fori_loop(0, _CPW, body, 0)
        plsc.subcore_barrier()
        pltpu.sync_copy(acc.at[pl.ds(sid * _RPT, _RPT)],
                        out_hbm.at[cid, pl.ds(sid * _RPT, _RPT)])

    return deg_kernel


def _make_spmm_kernel(d):
    @functools.partial(
        pl.kernel,
        mesh=_sc_mesh(),
        compiler_params=pltpu.CompilerParams(use_tc_tiling_on_sc=(d % 128 == 0)),
        out_type=jax.ShapeDtypeStruct((_NC, _N2, d), jnp.float32),
        scratch_types=[
            pltpu.VMEM((_CPW, _CHUNK), jnp.int32),
            pltpu.VMEM((_CPW, _CHUNK), jnp.int32),
            pltpu.VMEM((_CHUNK, d), jnp.float32),
            pltpu.VMEM_SHARED((_N2, d), jnp.float32),
            pltpu.SemaphoreType.DMA,
        ],
    )
    def spmm_kernel(g_hbm, src_hbm, dst_hbm, zeros_hbm, out_hbm,
                    src_v, dst_v, rows_v, acc, sem):
        cid = lax.axis_index("c")
        sid = lax.axis_index("s")
        wid = sid * _NC + cid
        pltpu.sync_copy(src_hbm.at[wid], src_v)
        pltpu.sync_copy(dst_hbm.at[wid], dst_v)
        pltpu.sync_copy(zeros_hbm.at[pl.ds(sid * _RPT, _RPT)],
                        acc.at[pl.ds(sid * _RPT, _RPT)])
        plsc.subcore_barrier()

        def body(j, carry):
            pltpu.async_copy(g_hbm.at[src_v.at[j]], rows_v, sem).wait()
            pltpu.sync_copy(rows_v, acc.at[dst_v.at[j]], add=True)
            return carry

        lax.fori_loop(0, _CPW, body, 0)
        plsc.subcore_barrier()
        pltpu.sync_copy(acc.at[pl.ds(sid * _RPT, _RPT)],
                        out_hbm.at[cid, pl.ds(sid * _RPT, _RPT)])

    return spmm_kernel


_deg_kernel = _make_deg_kernel()
_spmm128 = _make_spmm_kernel(_H1)
_spmm64 = _make_spmm_kernel(_H2)


def _g1_body(degp_ref, x_ref, w1_ref, g1_ref, dinv_ref):
    deg = degp_ref[0, :, 0:1] + degp_ref[1, :, 0:1] + 1.0
    dinv = lax.rsqrt(deg)
    h = jnp.dot(x_ref[...], w1_ref[...], preferred_element_type=jnp.float32)
    g1_ref[...] = h * dinv
    dinv_ref[...] = dinv


def _layer2_body(s1p_ref, g1_ref, dinv_ref, b1_ref, w2_ref, g2_ref):
    s = s1p_ref[0] + s1p_ref[1] + g1_ref[...]
    h1 = jnp.maximum(s * dinv_ref[...] + b1_ref[...], 0.0)
    g2_ref[...] = jnp.dot(h1, w2_ref[...],
                          preferred_element_type=jnp.float32) * dinv_ref[...]


def _pool_body(s2p_ref, g2_ref, dinv_ref, b2_ref, batch_ref, wfc_ref, bfc_ref,
               out_ref, pooled, counts):
    i = pl.program_id(0)

    @pl.when(i == 0)
    def _init():
        pooled[...] = jnp.zeros_like(pooled)
        counts[...] = jnp.zeros_like(counts)

    h2 = (s2p_ref[0] + s2p_ref[1] + g2_ref[...]) * dinv_ref[...] + b2_ref[...]
    gid = lax.broadcasted_iota(jnp.int32, (_G, _RB), 0)
    mask = jnp.where(gid == batch_ref[0], 1.0, 0.0)
    pooled[...] += jnp.dot(mask, h2, preferred_element_type=jnp.float32)
    counts[...] += jnp.sum(mask, axis=1, keepdims=True)

    @pl.when(i == pl.num_programs(0) - 1)
    def _fin():
        p = pooled[...] / jnp.maximum(counts[...], 1.0)
        out_ref[...] = jnp.dot(p, wfc_ref[...],
                               preferred_element_type=jnp.float32) + bfc_ref[...]


def kernel(x, edge_index, batch, W1, b1, W2, b2, Wfc, bfc):
    grid = _N // _RB

    src = edge_index[0]
    dst = edge_index[1]
    pad = jnp.arange(_EPAD - _E, dtype=jnp.int32) % 16
    src3 = jnp.concatenate([src, pad]).reshape(_NW, _CPW, _CHUNK)
    dst3 = jnp.concatenate([dst, _N + pad]).reshape(_NW, _CPW, _CHUNK)

    ones_dw = jnp.ones((_CHUNK, _DW), jnp.float32)
    zeros_dw = jnp.zeros((_N2, _DW), jnp.float32)
    zeros_h1 = jnp.zeros((_N2, _H1), jnp.float32)
    zeros_h2 = jnp.zeros((_N2, _H2), jnp.float32)

    degp = _deg_kernel(dst3, ones_dw, zeros_dw)

    g1, dinv = pl.pallas_call(
        _g1_body,
        grid=(grid,),
        in_specs=[
            pl.BlockSpec((_NC, _RB, _DW), lambda i: (0, i, 0)),
            pl.BlockSpec((_RB, _D), lambda i: (i, 0)),
            pl.BlockSpec((_D, _H1), lambda i: (0, 0)),
        ],
        out_specs=[
            pl.BlockSpec((_RB, _H1), lambda i: (i, 0)),
            pl.BlockSpec((_RB, 1), lambda i: (i, 0)),
        ],
        out_shape=[
            jax.ShapeDtypeStruct((_N, _H1), jnp.float32),
            jax.ShapeDtypeStruct((_N, 1), jnp.float32),
        ],
    )(degp, x, W1)

    s1p = _spmm128(g1, src3, dst3, zeros_h1)

    g2 = pl.pallas_call(
        _layer2_body,
        grid=(grid,),
        in_specs=[
            pl.BlockSpec((_NC, _RB, _H1), lambda i: (0, i, 0)),
            pl.BlockSpec((_RB, _H1), lambda i: (i, 0)),
            pl.BlockSpec((_RB, 1), lambda i: (i, 0)),
            pl.BlockSpec((1, _H1), lambda i: (0, 0)),
            pl.BlockSpec((_H1, _H2), lambda i: (0, 0)),
        ],
        out_specs=pl.BlockSpec((_RB, _H2), lambda i: (i, 0)),
        out_shape=jax.ShapeDtypeStruct((_N, _H2), jnp.float32),
    )(s1p, g1, dinv, b1.reshape(1, _H1), W2)

    s2p = _spmm64(g2, src3, dst3, zeros_h2)

    out = pl.pallas_call(
        _pool_body,
        grid=(grid,),
        in_specs=[
            pl.BlockSpec((_NC, _RB, _H2), lambda i: (0, i, 0)),
            pl.BlockSpec((_RB, _H2), lambda i: (i, 0)),
            pl.BlockSpec((_RB, 1), lambda i: (i, 0)),
            pl.BlockSpec((1, _H2), lambda i: (0, 0)),
            pl.BlockSpec((1, 1, _RB), lambda i: (i, 0, 0)),
            pl.BlockSpec((_H2, _C), lambda i: (0, 0)),
            pl.BlockSpec((1, _C), lambda i: (0, 0)),
        ],
        out_specs=pl.BlockSpec((_G, _C), lambda i: (0, 0)),
        out_shape=jax.ShapeDtypeStruct((_G, _C), jnp.float32),
        scratch_shapes=[
            pltpu.VMEM((_G, _H2), jnp.float32),
            pltpu.VMEM((_G, 1), jnp.float32),
        ],
    )(s2p, g2, dinv, b2.reshape(1, _H2), batch.reshape(grid, 1, _RB), Wfc,
      bfc.reshape(1, _C))

    return out


# speedup vs baseline: 30.4890x; 1.2257x over previous
import functools

import jax
import jax.numpy as jnp
from jax import lax
from jax.experimental import pallas as pl
from jax.experimental.pallas import tpu as pltpu
from jax.experimental.pallas import tpu_sc as plsc

_N = 10000
_E = 320000
_D = 128
_H1 = 128
_H2 = 64
_G = 64
_C = 10

_NC = 2
_NS = 16
_NW = _NC * _NS
_CHUNK = 128
_CPW = 80
_NPAIR = _CPW // 2
_EPAD = _NW * _CPW * _CHUNK
_DW = 16
_N2 = 10112
_RPT = _N2 // _NS

_RB = 2000


def _sc_mesh():
    return plsc.VectorSubcoreMesh(
        core_axis_name="c", subcore_axis_name="s",
        num_cores=_NC, num_subcores=_NS)


def _make_deg_kernel():
    @functools.partial(
        pl.kernel,
        mesh=_sc_mesh(),
        compiler_params=pltpu.CompilerParams(use_tc_tiling_on_sc=False),
        out_type=jax.ShapeDtypeStruct((_NC, _N2, _DW), jnp.float32),
        scratch_types=[
            pltpu.VMEM((_CPW, _CHUNK), jnp.int32),
            pltpu.VMEM((_CHUNK, _DW), jnp.float32),
            pltpu.VMEM_SHARED((_N2, _DW), jnp.float32),
            pltpu.SemaphoreType.DMA,
        ],
    )
    def deg_kernel(dst_hbm, ones_hbm, zeros_hbm, out_hbm, dst_v, ones_v, acc, sem):
        cid = lax.axis_index("c")
        sid = lax.axis_index("s")
        wid = sid * _NC + cid
        pltpu.sync_copy(dst_hbm.at[wid], dst_v)
        pltpu.sync_copy(ones_hbm, ones_v)
        pltpu.sync_copy(zeros_hbm.at[pl.ds(sid * _RPT, _RPT)],
                        acc.at[pl.ds(sid * _RPT, _RPT)])
        plsc.subcore_barrier()

        def body(j, carry):
            pltpu.sync_copy(ones_v, acc.at[dst_v.at[j]], add=True)
            return carry

        lax.fori_loop(0, _CPW, body, 0)
        plsc.subcore_barrier()
        pltpu.sync_copy(acc.at[pl.ds(sid * _RPT, _RPT)],
                        out_hbm.at[cid, pl.ds(sid * _RPT, _RPT)])

    return deg_kernel


def _make_spmm_kernel(d):
    @functools.partial(
        pl.kernel,
        mesh=_sc_mesh(),
        compiler_params=pltpu.CompilerParams(use_tc_tiling_on_sc=(d % 128 == 0)),
        out_type=jax.ShapeDtypeStruct((_NC, _N2, d), jnp.float32),
        scratch_types=[
            pltpu.VMEM((_CPW, _CHUNK), jnp.int32),
            pltpu.VMEM((_NPAIR, _CHUNK), jnp.int32),
            pltpu.VMEM((_CHUNK, d), jnp.float32),
            pltpu.VMEM((_CHUNK, d), jnp.float32),
            pltpu.VMEM_SHARED((_N2, d), jnp.float32),
            pltpu.SemaphoreType.DMA,
            pltpu.SemaphoreType.DMA,
            pltpu.SemaphoreType.DMA,
            pltpu.SemaphoreType.DMA,
        ],
    )
    def spmm_kernel(g_hbm, src_hbm, dst_hbm, out_hbm,
                    src_v, dst_v, rows0, rows1, acc,
                    gsem0, gsem1, ssem0, ssem1):
        cid = lax.axis_index("c")
        sid = lax.axis_index("s")
        wid = sid * _NC + cid
        z16 = jnp.zeros((16,), jnp.float32)
        for r in range(8):
            for c in range(d // 16):
                rows0[r, pl.ds(c * 16, 16)] = z16

        def zb(i, carry):
            pltpu.sync_copy(rows0.at[pl.ds(0, 8)],
                            acc.at[pl.ds(sid * _RPT + i * 8, 8)])
            return carry

        lax.fori_loop(0, _RPT // 8, zb, 0)
        pltpu.sync_copy(src_hbm.at[wid], src_v)
        pltpu.sync_copy(dst_hbm.at[wid, pl.ds(0, _NPAIR)], dst_v)
        plsc.subcore_barrier()

        def fire_g(j, buf, sem):
            pltpu.async_copy(g_hbm.at[src_v.at[j]], buf, sem)

        def wait_g(j, buf, sem):
            pltpu.make_async_copy(g_hbm.at[src_v.at[j]], buf, sem).wait()

        def fire_s(r, buf, sem):
            pltpu.async_copy(buf, acc.at[dst_v.at[r]], sem, add=True)

        def wait_s(buf, sem):
            pltpu.make_async_copy(buf, acc.at[dst_v.at[0]], sem).wait()

        fire_g(0, rows0, gsem0)

        def body(m, carry):
            j0 = 2 * m
            j1 = j0 + 1
            r0 = lax.rem(j0, _NPAIR)
            r1 = lax.rem(j1, _NPAIR)

            wait_g(j0, rows0, gsem0)

            @pl.when(m > 0)
            def _():
                wait_s(rows1, ssem1)

            @pl.when(m == _NPAIR // 2)
            def _():
                pltpu.sync_copy(dst_hbm.at[wid, pl.ds(_NPAIR, _NPAIR)], dst_v)

            fire_s(r0, rows0, ssem0)
            fire_g(j1, rows1, gsem1)
            wait_g(j1, rows1, gsem1)
            fire_s(r1, rows1, ssem1)

            @pl.when(m < _NPAIR - 1)
            def _():
                wait_s(rows0, ssem0)
                fire_g(j0 + 2, rows0, gsem0)

            return carry

        lax.fori_loop(0, _NPAIR, body, 0)
        wait_s(rows0, ssem0)
        wait_s(rows1, ssem1)
        plsc.subcore_barrier()
        pltpu.sync_copy(acc.at[pl.ds(sid * _RPT, _RPT)],
                        out_hbm.at[cid, pl.ds(sid * _RPT, _RPT)])

    return spmm_kernel


_deg_kernel = _make_deg_kernel()
_spmm128 = _make_spmm_kernel(_H1)
_spmm64 = _make_spmm_kernel(_H2)


def _g1_body(degp_ref, x_ref, w1_ref, g1_ref, dinv_ref):
    deg = degp_ref[0, :, 0:1] + degp_ref[1, :, 0:1] + 1.0
    dinv = lax.rsqrt(deg)
    h = jnp.dot(x_ref[...], w1_ref[...], preferred_element_type=jnp.float32)
    g1_ref[...] = h * dinv
    dinv_ref[...] = dinv


def _layer2_body(s1p_ref, g1_ref, dinv_ref, b1_ref, w2_ref, g2_ref):
    s = s1p_ref[0] + s1p_ref[1] + g1_ref[...]
    h1 = jnp.maximum(s * dinv_ref[...] + b1_ref[...], 0.0)
    g2_ref[...] = jnp.dot(h1, w2_ref[...],
                          preferred_element_type=jnp.float32) * dinv_ref[...]


def _pool_body(s2p_ref, g2_ref, dinv_ref, b2_ref, batch_ref, wfc_ref, bfc_ref,
               out_ref, pooled, counts):
    i = pl.program_id(0)

    @pl.when(i == 0)
    def _init():
        pooled[...] = jnp.zeros_like(pooled)
        counts[...] = jnp.zeros_like(counts)

    h2 = (s2p_ref[0] + s2p_ref[1] + g2_ref[...]) * dinv_ref[...] + b2_ref[...]
    gid = lax.broadcasted_iota(jnp.int32, (_G, _RB), 0)
    mask = jnp.where(gid == batch_ref[0], 1.0, 0.0)
    pooled[...] += jnp.dot(mask, h2, preferred_element_type=jnp.float32)
    counts[...] += jnp.sum(mask, axis=1, keepdims=True)

    @pl.when(i == pl.num_programs(0) - 1)
    def _fin():
        p = pooled[...] / jnp.maximum(counts[...], 1.0)
        out_ref[...] = jnp.dot(p, wfc_ref[...],
                               preferred_element_type=jnp.float32) + bfc_ref[...]


def kernel(x, edge_index, batch, W1, b1, W2, b2, Wfc, bfc):
    grid = _N // _RB

    src = edge_index[0]
    dst = edge_index[1]
    pad = jnp.arange(_EPAD - _E, dtype=jnp.int32)
    src3 = jnp.concatenate([src, pad % 512]).reshape(_NW, _CPW, _CHUNK)
    dst3 = jnp.concatenate([dst, _N + pad % (_N2 - _N)]).reshape(_NW, _CPW, _CHUNK)

    ones_dw = jnp.ones((_CHUNK, _DW), jnp.float32)
    zeros_dw = jnp.zeros((_N2, _DW), jnp.float32)

    degp = _deg_kernel(dst3, ones_dw, zeros_dw)

    g1, dinv = pl.pallas_call(
        _g1_body,
        grid=(grid,),
        in_specs=[
            pl.BlockSpec((_NC, _RB, _DW), lambda i: (0, i, 0)),
            pl.BlockSpec((_RB, _D), lambda i: (i, 0)),
            pl.BlockSpec((_D, _H1), lambda i: (0, 0)),
        ],
        out_specs=[
            pl.BlockSpec((_RB, _H1), lambda i: (i, 0)),
            pl.BlockSpec((_RB, 1), lambda i: (i, 0)),
        ],
        out_shape=[
            jax.ShapeDtypeStruct((_N, _H1), jnp.float32),
            jax.ShapeDtypeStruct((_N, 1), jnp.float32),
        ],
    )(degp, x, W1)

    s1p = _spmm128(g1, src3, dst3)

    g2 = pl.pallas_call(
        _layer2_body,
        grid=(grid,),
        in_specs=[
            pl.BlockSpec((_NC, _RB, _H1), lambda i: (0, i, 0)),
            pl.BlockSpec((_RB, _H1), lambda i: (i, 0)),
            pl.BlockSpec((_RB, 1), lambda i: (i, 0)),
            pl.BlockSpec((1, _H1), lambda i: (0, 0)),
            pl.BlockSpec((_H1, _H2), lambda i: (0, 0)),
        ],
        out_specs=pl.BlockSpec((_RB, _H2), lambda i: (i, 0)),
        out_shape=jax.ShapeDtypeStruct((_N, _H2), jnp.float32),
    )(s1p, g1, dinv, b1.reshape(1, _H1), W2)

    s2p = _spmm64(g2, src3, dst3)

    out = pl.pallas_call(
        _pool_body,
        grid=(grid,),
        in_specs=[
            pl.BlockSpec((_NC, _RB, _H2), lambda i: (0, i, 0)),
            pl.BlockSpec((_RB, _H2), lambda i: (i, 0)),
            pl.BlockSpec((_RB, 1), lambda i: (i, 0)),
            pl.BlockSpec((1, _H2), lambda i: (0, 0)),
            pl.BlockSpec((1, 1, _RB), lambda i: (i, 0, 0)),
            pl.BlockSpec((_H2, _C), lambda i: (0, 0)),
            pl.BlockSpec((1, _C), lambda i: (0, 0)),
        ],
        out_specs=pl.BlockSpec((_G, _C), lambda i: (0, 0)),
        out_shape=jax.ShapeDtypeStruct((_G, _C), jnp.float32),
        scratch_shapes=[
            pltpu.VMEM((_G, _H2), jnp.float32),
            pltpu.VMEM((_G, 1), jnp.float32),
        ],
    )(s2p, g2, dinv, b2.reshape(1, _H2), batch.reshape(grid, 1, _RB), Wfc,
      bfc.reshape(1, _C))

    return out


# speedup vs baseline: 34.0219x; 1.1159x over previous
import functools

import jax
import jax.numpy as jnp
from jax import lax
from jax.experimental import pallas as pl
from jax.experimental.pallas import tpu as pltpu
from jax.experimental.pallas import tpu_sc as plsc

_N = 10000
_E = 320000
_D = 128
_H1 = 128
_H2 = 64
_G = 64
_C = 10

_NC = 2
_NS = 16
_NW = _NC * _NS
_CHUNK = 128
_CPW = 80
_NPAIR = _CPW // 2
_EPAD = _NW * _CPW * _CHUNK
_DW = 16
_N2 = 10112
_RPT = _N2 // _NS

_RB = 2000


def _sc_mesh():
    return plsc.VectorSubcoreMesh(
        core_axis_name="c", subcore_axis_name="s",
        num_cores=_NC, num_subcores=_NS)


def _make_deg_kernel():
    @functools.partial(
        pl.kernel,
        mesh=_sc_mesh(),
        compiler_params=pltpu.CompilerParams(use_tc_tiling_on_sc=False),
        out_type=jax.ShapeDtypeStruct((_NC, _N2, _DW), jnp.float32),
        scratch_types=[
            pltpu.VMEM((_CPW, _CHUNK), jnp.int32),
            pltpu.VMEM((_CHUNK, _DW), jnp.float32),
            pltpu.VMEM_SHARED((_N2, _DW), jnp.float32),
            pltpu.SemaphoreType.DMA,
        ],
    )
    def deg_kernel(dst_hbm, ones_hbm, zeros_hbm, out_hbm, dst_v, ones_v, acc, sem):
        cid = lax.axis_index("c")
        sid = lax.axis_index("s")
        wid = sid * _NC + cid
        pltpu.sync_copy(dst_hbm.at[wid], dst_v)
        pltpu.sync_copy(ones_hbm, ones_v)
        pltpu.sync_copy(zeros_hbm.at[pl.ds(sid * _RPT, _RPT)],
                        acc.at[pl.ds(sid * _RPT, _RPT)])
        plsc.subcore_barrier()

        def body(j, carry):
            pltpu.sync_copy(ones_v, acc.at[dst_v.at[j]], add=True)
            return carry

        lax.fori_loop(0, _CPW, body, 0)
        plsc.subcore_barrier()
        pltpu.sync_copy(acc.at[pl.ds(sid * _RPT, _RPT)],
                        out_hbm.at[cid, pl.ds(sid * _RPT, _RPT)])

    return deg_kernel


def _make_spmm_kernel(d):
    @functools.partial(
        pl.kernel,
        mesh=_sc_mesh(),
        compiler_params=pltpu.CompilerParams(use_tc_tiling_on_sc=(d % 128 == 0)),
        out_type=jax.ShapeDtypeStruct((_NC, _N2, d), jnp.float32),
        scratch_types=[
            pltpu.VMEM((_CPW, _CHUNK), jnp.int32),
            pltpu.VMEM((_NPAIR, _CHUNK), jnp.int32),
            pltpu.VMEM((_CHUNK, d), jnp.float32),
            pltpu.VMEM((_CHUNK, d), jnp.float32),
            pltpu.VMEM_SHARED((_N2, d), jnp.float32),
            pltpu.SemaphoreType.DMA,
            pltpu.SemaphoreType.DMA,
            pltpu.SemaphoreType.DMA,
            pltpu.SemaphoreType.DMA,
        ],
    )
    def spmm_kernel(g_hbm, src_hbm, dst_hbm, out_hbm,
                    src_v, dst_v, rows0, rows1, acc,
                    gsem0, gsem1, ssem0, ssem1):
        cid = lax.axis_index("c")
        sid = lax.axis_index("s")
        wid = sid * _NC + cid
        z16 = jnp.zeros((16,), jnp.float32)
        for r in range(8):
            for c in range(d // 16):
                rows0[r, pl.ds(c * 16, 16)] = z16

        def zb(i, carry):
            pltpu.sync_copy(rows0.at[pl.ds(0, 8)],
                            acc.at[pl.ds(sid * _RPT + i * 8, 8)])
            return carry

        lax.fori_loop(0, _RPT // 8, zb, 0)
        pltpu.sync_copy(src_hbm.at[wid], src_v)
        pltpu.sync_copy(dst_hbm.at[wid, pl.ds(0, _NPAIR)], dst_v)
        plsc.subcore_barrier()

        def fire_g(j, buf, sem):
            pltpu.async_copy(g_hbm.at[src_v.at[j]], buf, sem)

        def wait_g(j, buf, sem):
            pltpu.make_async_copy(g_hbm.at[src_v.at[j]], buf, sem).wait()

        def fire_s(r, buf, sem):
            pltpu.async_copy(buf, acc.at[dst_v.at[r]], sem, add=True)

        def wait_s(buf, sem):
            pltpu.make_async_copy(buf, acc.at[dst_v.at[0]], sem).wait()

        fire_g(0, rows0, gsem0)

        def body(m, carry):
            j0 = 2 * m
            j1 = j0 + 1
            r0 = lax.rem(j0, _NPAIR)
            r1 = lax.rem(j1, _NPAIR)

            wait_g(j0, rows0, gsem0)

            @pl.when(m > 0)
            def _():
                wait_s(rows1, ssem1)

            @pl.when(m == _NPAIR // 2)
            def _():
                pltpu.sync_copy(dst_hbm.at[wid, pl.ds(_NPAIR, _NPAIR)], dst_v)

            fire_s(r0, rows0, ssem0)
            fire_g(j1, rows1, gsem1)
            wait_g(j1, rows1, gsem1)
            fire_s(r1, rows1, ssem1)

            @pl.when(m < _NPAIR - 1)
            def _():
                wait_s(rows0, ssem0)
                fire_g(j0 + 2, rows0, gsem0)

            return carry

        lax.fori_loop(0, _NPAIR, body, 0)
        wait_s(rows0, ssem0)
        wait_s(rows1, ssem1)
        plsc.subcore_barrier()
        pltpu.sync_copy(acc.at[pl.ds(sid * _RPT, _RPT)],
                        out_hbm.at[cid, pl.ds(sid * _RPT, _RPT)])

    return spmm_kernel


def _make_spmm_ring3_kernel(d):
    @functools.partial(
        pl.kernel,
        mesh=_sc_mesh(),
        compiler_params=pltpu.CompilerParams(use_tc_tiling_on_sc=False),
        out_type=jax.ShapeDtypeStruct((_NC, _N2, d), jnp.float32),
        scratch_types=[
            pltpu.VMEM((_CPW, _CHUNK), jnp.int32),
            pltpu.VMEM((_CPW, _CHUNK), jnp.int32),
            pltpu.VMEM((_CHUNK, d), jnp.float32),
            pltpu.VMEM((_CHUNK, d), jnp.float32),
            pltpu.VMEM((_CHUNK, d), jnp.float32),
            pltpu.VMEM_SHARED((_N2, d), jnp.float32),
            pltpu.SemaphoreType.DMA,
            pltpu.SemaphoreType.DMA,
            pltpu.SemaphoreType.DMA,
            pltpu.SemaphoreType.DMA,
            pltpu.SemaphoreType.DMA,
            pltpu.SemaphoreType.DMA,
        ],
    )
    def spmm_kernel(g_hbm, src_hbm, dst_hbm, out_hbm,
                    src_v, dst_v, b0, b1, b2, acc,
                    g0, g1, g2, s0, s1, s2):
        cid = lax.axis_index("c")
        sid = lax.axis_index("s")
        wid = sid * _NC + cid
        bufs = (b0, b1, b2)
        gsems = (g0, g1, g2)
        ssems = (s0, s1, s2)
        z16 = jnp.zeros((16,), jnp.float32)
        for r in range(8):
            for c in range(d // 16):
                b0[r, pl.ds(c * 16, 16)] = z16

        def zb(i, carry):
            pltpu.sync_copy(b0.at[pl.ds(0, 8)],
                            acc.at[pl.ds(sid * _RPT + i * 8, 8)])
            return carry

        lax.fori_loop(0, _RPT // 8, zb, 0)
        pltpu.sync_copy(src_hbm.at[wid], src_v)
        pltpu.sync_copy(dst_hbm.at[wid], dst_v)
        plsc.subcore_barrier()

        def fire_g(j, b):
            pltpu.async_copy(g_hbm.at[src_v.at[j]], bufs[b], gsems[b])

        def wait_g(j, b):
            pltpu.make_async_copy(g_hbm.at[src_v.at[j]], bufs[b], gsems[b]).wait()

        def fire_s(j, b):
            pltpu.async_copy(bufs[b], acc.at[dst_v.at[j]], ssems[b], add=True)

        def wait_s(b):
            pltpu.make_async_copy(bufs[b], acc.at[dst_v.at[0]], ssems[b]).wait()

        fire_g(0, 0)
        fire_g(1, 1)

        ntri = _CPW // 3

        def body(m, carry):
            j0 = 3 * m

            wait_g(j0, 0)
            fire_s(j0, 0)

            @pl.when(m > 0)
            def _():
                wait_s(2)

            fire_g(j0 + 2, 2)

            wait_g(j0 + 1, 1)
            fire_s(j0 + 1, 1)
            wait_s(0)
            fire_g(j0 + 3, 0)

            wait_g(j0 + 2, 2)
            fire_s(j0 + 2, 2)
            wait_s(1)
            fire_g(j0 + 4, 1)
            return carry

        lax.fori_loop(0, ntri, body, 0)
        wait_g(_CPW - 2, 0)
        fire_s(_CPW - 2, 0)
        wait_g(_CPW - 1, 1)
        fire_s(_CPW - 1, 1)
        wait_s(2)
        wait_s(0)
        wait_s(1)
        plsc.subcore_barrier()
        pltpu.sync_copy(acc.at[pl.ds(sid * _RPT, _RPT)],
                        out_hbm.at[cid, pl.ds(sid * _RPT, _RPT)])

    return spmm_kernel


_deg_kernel = _make_deg_kernel()
_spmm128 = _make_spmm_kernel(_H1)
_spmm64 = _make_spmm_ring3_kernel(_H2)


def _g1_body(degp_ref, x_ref, w1_ref, g1_ref, dinv_ref):
    deg = degp_ref[0, :, 0:1] + degp_ref[1, :, 0:1] + 1.0
    dinv = lax.rsqrt(deg)
    h = jnp.dot(x_ref[...], w1_ref[...], preferred_element_type=jnp.float32)
    g1_ref[...] = h * dinv
    dinv_ref[...] = dinv


def _layer2_body(s1p_ref, g1_ref, dinv_ref, b1_ref, w2_ref, g2_ref):
    s = s1p_ref[0] + s1p_ref[1] + g1_ref[...]
    h1 = jnp.maximum(s * dinv_ref[...] + b1_ref[...], 0.0)
    g2_ref[...] = jnp.dot(h1, w2_ref[...],
                          preferred_element_type=jnp.float32) * dinv_ref[...]


def _pool_body(s2p_ref, g2_ref, dinv_ref, b2_ref, batch_ref, wfc_ref, bfc_ref,
               out_ref, pooled, counts):
    i = pl.program_id(0)

    @pl.when(i == 0)
    def _init():
        pooled[...] = jnp.zeros_like(pooled)
        counts[...] = jnp.zeros_like(counts)

    h2 = (s2p_ref[0] + s2p_ref[1] + g2_ref[...]) * dinv_ref[...] + b2_ref[...]
    gid = lax.broadcasted_iota(jnp.int32, (_G, _RB), 0)
    mask = jnp.where(gid == batch_ref[0], 1.0, 0.0)
    pooled[...] += jnp.dot(mask, h2, preferred_element_type=jnp.float32)
    counts[...] += jnp.sum(mask, axis=1, keepdims=True)

    @pl.when(i == pl.num_programs(0) - 1)
    def _fin():
        p = pooled[...] / jnp.maximum(counts[...], 1.0)
        out_ref[...] = jnp.dot(p, wfc_ref[...],
                               preferred_element_type=jnp.float32) + bfc_ref[...]


def kernel(x, edge_index, batch, W1, b1, W2, b2, Wfc, bfc):
    grid = _N // _RB

    src = edge_index[0]
    dst = edge_index[1]
    pad = jnp.arange(_EPAD - _E, dtype=jnp.int32)
    src3 = jnp.concatenate([src, pad % 512]).reshape(_NW, _CPW, _CHUNK)
    dst3 = jnp.concatenate([dst, _N + pad % (_N2 - _N)]).reshape(_NW, _CPW, _CHUNK)

    ones_dw = jnp.ones((_CHUNK, _DW), jnp.float32)
    zeros_dw = jnp.zeros((_N2, _DW), jnp.float32)

    degp = _deg_kernel(dst3, ones_dw, zeros_dw)

    g1, dinv = pl.pallas_call(
        _g1_body,
        grid=(grid,),
        in_specs=[
            pl.BlockSpec((_NC, _RB, _DW), lambda i: (0, i, 0)),
            pl.BlockSpec((_RB, _D), lambda i: (i, 0)),
            pl.BlockSpec((_D, _H1), lambda i: (0, 0)),
        ],
        out_specs=[
            pl.BlockSpec((_RB, _H1), lambda i: (i, 0)),
            pl.BlockSpec((_RB, 1), lambda i: (i, 0)),
        ],
        out_shape=[
            jax.ShapeDtypeStruct((_N, _H1), jnp.float32),
            jax.ShapeDtypeStruct((_N, 1), jnp.float32),
        ],
    )(degp, x, W1)

    s1p = _spmm128(g1, src3, dst3)

    g2 = pl.pallas_call(
        _layer2_body,
        grid=(grid,),
        in_specs=[
            pl.BlockSpec((_NC, _RB, _H1), lambda i: (0, i, 0)),
            pl.BlockSpec((_RB, _H1), lambda i: (i, 0)),
            pl.BlockSpec((_RB, 1), lambda i: (i, 0)),
            pl.BlockSpec((1, _H1), lambda i: (0, 0)),
            pl.BlockSpec((_H1, _H2), lambda i: (0, 0)),
        ],
        out_specs=pl.BlockSpec((_RB, _H2), lambda i: (i, 0)),
        out_shape=jax.ShapeDtypeStruct((_N, _H2), jnp.float32),
    )(s1p, g1, dinv, b1.reshape(1, _H1), W2)

    s2p = _spmm64(g2, src3, dst3)

    out = pl.pallas_call(
        _pool_body,
        grid=(grid,),
        in_specs=[
            pl.BlockSpec((_NC, _RB, _H2), lambda i: (0, i, 0)),
            pl.BlockSpec((_RB, _H2), lambda i: (i, 0)),
            pl.BlockSpec((_RB, 1), lambda i: (i, 0)),
            pl.BlockSpec((1, _H2), lambda i: (0, 0)),
            pl.BlockSpec((1, 1, _RB), lambda i: (i, 0, 0)),
            pl.BlockSpec((_H2, _C), lambda i: (0, 0)),
            pl.BlockSpec((1, _C), lambda i: (0, 0)),
        ],
        out_specs=pl.BlockSpec((_G, _C), lambda i: (0, 0)),
        out_shape=jax.ShapeDtypeStruct((_G, _C), jnp.float32),
        scratch_shapes=[
            pltpu.VMEM((_G, _H2), jnp.float32),
            pltpu.VMEM((_G, 1), jnp.float32),
        ],
    )(s2p, g2, dinv, b2.reshape(1, _H2), batch.reshape(grid, 1, _RB), Wfc,
      bfc.reshape(1, _C))

    return out


# speedup vs baseline: 36.9027x; 1.0847x over previous
import functools

import jax
import jax.numpy as jnp
from jax import lax
from jax.experimental import pallas as pl
from jax.experimental.pallas import tpu as pltpu
from jax.experimental.pallas import tpu_sc as plsc

_N = 10000
_E = 320000
_D = 128
_H1 = 128
_H2 = 64
_G = 64
_C = 10

_NC = 2
_NS = 16
_NW = _NC * _NS
_CHUNK = 80
_CPW = _E // (_NW * _CHUNK)
_NTRI = (_CPW - 2) // 3
_DW = 16
_RPT = _N // _NS

_RB = 2000


def _sc_mesh():
    return plsc.VectorSubcoreMesh(
        core_axis_name="c", subcore_axis_name="s",
        num_cores=_NC, num_subcores=_NS)


def _zero_acc(buf, acc, sid, d):
    z16 = jnp.zeros((16,), jnp.float32)
    for r in range(8):
        for c in range(d // 16):
            buf[r, pl.ds(c * 16, 16)] = z16

    def zb(i, carry):
        pltpu.sync_copy(buf.at[pl.ds(0, 8)],
                        acc.at[pl.ds(sid * _RPT + i * 8, 8)])
        return carry

    lax.fori_loop(0, _RPT // 8, zb, 0)
    pltpu.sync_copy(buf.at[pl.ds(0, 1)],
                    acc.at[pl.ds(sid * _RPT + (_RPT // 8) * 8, 1)])


def _make_deg_kernel():
    @functools.partial(
        pl.kernel,
        mesh=_sc_mesh(),
        compiler_params=pltpu.CompilerParams(use_tc_tiling_on_sc=False),
        out_type=jax.ShapeDtypeStruct((_NC, _N, _DW), jnp.float32),
        scratch_types=[
            pltpu.VMEM((_CPW, _CHUNK), jnp.int32),
            pltpu.VMEM((_CHUNK, _DW), jnp.float32),
            pltpu.VMEM_SHARED((_N, _DW), jnp.float32),
            pltpu.SemaphoreType.DMA,
        ],
    )
    def deg_kernel(dst_hbm, ones_hbm, out_hbm, dst_v, ones_v, acc, sem):
        cid = lax.axis_index("c")
        sid = lax.axis_index("s")
        wid = sid * _NC + cid
        pltpu.sync_copy(dst_hbm.at[wid], dst_v)
        pltpu.sync_copy(ones_hbm, ones_v)
        _zero_acc(ones_v, acc, sid, _DW)
        pltpu.sync_copy(ones_hbm, ones_v)
        plsc.subcore_barrier()

        def fire_s(j):
            pltpu.async_copy(ones_v, acc.at[dst_v.at[j]], sem, add=True)

        def wait_s():
            pltpu.make_async_copy(ones_v, acc.at[dst_v.at[0]], sem).wait()

        def body(j, carry):
            fire_s(j)

            @pl.when(j > 0)
            def _():
                wait_s()

            return carry

        lax.fori_loop(0, _CPW, body, 0)
        wait_s()
        plsc.subcore_barrier()
        pltpu.sync_copy(acc.at[pl.ds(sid * _RPT, _RPT)],
                        out_hbm.at[cid, pl.ds(sid * _RPT, _RPT)])

    return deg_kernel


def _make_spmm_kernel(d):
    @functools.partial(
        pl.kernel,
        mesh=_sc_mesh(),
        compiler_params=pltpu.CompilerParams(use_tc_tiling_on_sc=False),
        out_type=jax.ShapeDtypeStruct((_NC, _N, d), jnp.float32),
        scratch_types=[
            pltpu.VMEM((_CPW, _CHUNK), jnp.int32),
            pltpu.VMEM((_CPW, _CHUNK), jnp.int32),
            pltpu.VMEM((_CHUNK, d), jnp.float32),
            pltpu.VMEM((_CHUNK, d), jnp.float32),
            pltpu.VMEM((_CHUNK, d), jnp.float32),
            pltpu.VMEM_SHARED((_N, d), jnp.float32),
            pltpu.SemaphoreType.DMA,
            pltpu.SemaphoreType.DMA,
            pltpu.SemaphoreType.DMA,
            pltpu.SemaphoreType.DMA,
            pltpu.SemaphoreType.DMA,
            pltpu.SemaphoreType.DMA,
        ],
    )
    def spmm_kernel(g_hbm, src_hbm, dst_hbm, out_hbm,
                    src_v, dst_v, b0, b1, b2, acc,
                    g0, g1, g2, s0, s1, s2):
        cid = lax.axis_index("c")
        sid = lax.axis_index("s")
        wid = sid * _NC + cid
        bufs = (b0, b1, b2)
        gsems = (g0, g1, g2)
        ssems = (s0, s1, s2)
        _zero_acc(b0, acc, sid, d)
        pltpu.sync_copy(src_hbm.at[wid], src_v)
        pltpu.sync_copy(dst_hbm.at[wid], dst_v)
        plsc.subcore_barrier()

        def fire_g(j, b):
            pltpu.async_copy(g_hbm.at[src_v.at[j]], bufs[b], gsems[b])

        def wait_g(j, b):
            pltpu.make_async_copy(g_hbm.at[src_v.at[j]], bufs[b], gsems[b]).wait()

        def fire_s(j, b):
            pltpu.async_copy(bufs[b], acc.at[dst_v.at[j]], ssems[b], add=True)

        def wait_s(b):
            pltpu.make_async_copy(bufs[b], acc.at[dst_v.at[0]], ssems[b]).wait()

        fire_g(0, 0)
        fire_g(1, 1)

        def body(m, carry):
            j0 = 3 * m

            wait_g(j0, 0)
            fire_s(j0, 0)

            @pl.when(m > 0)
            def _():
                wait_s(2)

            fire_g(j0 + 2, 2)

            wait_g(j0 + 1, 1)
            fire_s(j0 + 1, 1)
            wait_s(0)
            fire_g(j0 + 3, 0)

            wait_g(j0 + 2, 2)
            fire_s(j0 + 2, 2)
            wait_s(1)
            fire_g(j0 + 4, 1)
            return carry

        lax.fori_loop(0, _NTRI, body, 0)
        wait_g(_CPW - 2, 0)
        fire_s(_CPW - 2, 0)
        wait_g(_CPW - 1, 1)
        fire_s(_CPW - 1, 1)
        wait_s(2)
        wait_s(0)
        wait_s(1)
        plsc.subcore_barrier()
        pltpu.sync_copy(acc.at[pl.ds(sid * _RPT, _RPT)],
                        out_hbm.at[cid, pl.ds(sid * _RPT, _RPT)])

    return spmm_kernel


_deg_kernel = _make_deg_kernel()
_spmm128 = _make_spmm_kernel(_H1)
_spmm64 = _make_spmm_kernel(_H2)


def _g1_body(degp_ref, x_ref, w1_ref, g1_ref, dinv_ref):
    deg = degp_ref[0, :, 0:1] + degp_ref[1, :, 0:1] + 1.0
    dinv = lax.rsqrt(deg)
    h = jnp.dot(x_ref[...], w1_ref[...], preferred_element_type=jnp.float32)
    g1_ref[...] = h * dinv
    dinv_ref[...] = dinv


def _layer2_body(s1p_ref, g1_ref, dinv_ref, b1_ref, w2_ref, g2_ref):
    s = s1p_ref[0] + s1p_ref[1] + g1_ref[...]
    h1 = jnp.maximum(s * dinv_ref[...] + b1_ref[...], 0.0)
    g2_ref[...] = jnp.dot(h1, w2_ref[...],
                          preferred_element_type=jnp.float32) * dinv_ref[...]


def _pool_body(s2p_ref, g2_ref, dinv_ref, b2_ref, batch_ref, wfc_ref, bfc_ref,
               out_ref, pooled, counts):
    i = pl.program_id(0)

    @pl.when(i == 0)
    def _init():
        pooled[...] = jnp.zeros_like(pooled)
        counts[...] = jnp.zeros_like(counts)

    h2 = (s2p_ref[0] + s2p_ref[1] + g2_ref[...]) * dinv_ref[...] + b2_ref[...]
    gid = lax.broadcasted_iota(jnp.int32, (_G, _RB), 0)
    mask = jnp.where(gid == batch_ref[0], 1.0, 0.0)
    pooled[...] += jnp.dot(mask, h2, preferred_element_type=jnp.float32)
    counts[...] += jnp.sum(mask, axis=1, keepdims=True)

    @pl.when(i == pl.num_programs(0) - 1)
    def _fin():
        p = pooled[...] / jnp.maximum(counts[...], 1.0)
        out_ref[...] = jnp.dot(p, wfc_ref[...],
                               preferred_element_type=jnp.float32) + bfc_ref[...]


def kernel(x, edge_index, batch, W1, b1, W2, b2, Wfc, bfc):
    grid = _N // _RB

    src3 = edge_index[0].reshape(_NW, _CPW, _CHUNK)
    dst3 = edge_index[1].reshape(_NW, _CPW, _CHUNK)

    ones_dw = jnp.ones((_CHUNK, _DW), jnp.float32)

    degp = _deg_kernel(dst3, ones_dw)

    g1, dinv = pl.pallas_call(
        _g1_body,
        grid=(grid,),
        in_specs=[
            pl.BlockSpec((_NC, _RB, _DW), lambda i: (0, i, 0)),
            pl.BlockSpec((_RB, _D), lambda i: (i, 0)),
            pl.BlockSpec((_D, _H1), lambda i: (0, 0)),
        ],
        out_specs=[
            pl.BlockSpec((_RB, _H1), lambda i: (i, 0)),
            pl.BlockSpec((_RB, 1), lambda i: (i, 0)),
        ],
        out_shape=[
            jax.ShapeDtypeStruct((_N, _H1), jnp.float32),
            jax.ShapeDtypeStruct((_N, 1), jnp.float32),
        ],
    )(degp, x, W1)

    s1p = _spmm128(g1, src3, dst3)

    g2 = pl.pallas_call(
        _layer2_body,
        grid=(grid,),
        in_specs=[
            pl.BlockSpec((_NC, _RB, _H1), lambda i: (0, i, 0)),
            pl.BlockSpec((_RB, _H1), lambda i: (i, 0)),
            pl.BlockSpec((_RB, 1), lambda i: (i, 0)),
            pl.BlockSpec((1, _H1), lambda i: (0, 0)),
            pl.BlockSpec((_H1, _H2), lambda i: (0, 0)),
        ],
        out_specs=pl.BlockSpec((_RB, _H2), lambda i: (i, 0)),
        out_shape=jax.ShapeDtypeStruct((_N, _H2), jnp.float32),
    )(s1p, g1, dinv, b1.reshape(1, _H1), W2)

    s2p = _spmm64(g2, src3, dst3)

    out = pl.pallas_call(
        _pool_body,
        grid=(grid,),
        in_specs=[
            pl.BlockSpec((_NC, _RB, _H2), lambda i: (0, i, 0)),
            pl.BlockSpec((_RB, _H2), lambda i: (i, 0)),
            pl.BlockSpec((_RB, 1), lambda i: (i, 0)),
            pl.BlockSpec((1, _H2), lambda i: (0, 0)),
            pl.BlockSpec((1, 1, _RB), lambda i: (i, 0, 0)),
            pl.BlockSpec((_H2, _C), lambda i: (0, 0)),
            pl.BlockSpec((1, _C), lambda i: (0, 0)),
        ],
        out_specs=pl.BlockSpec((_G, _C), lambda i: (0, 0)),
        out_shape=jax.ShapeDtypeStruct((_G, _C), jnp.float32),
        scratch_shapes=[
            pltpu.VMEM((_G, _H2), jnp.float32),
            pltpu.VMEM((_G, 1), jnp.float32),
        ],
    )(s2p, g2, dinv, b2.reshape(1, _H2), batch.reshape(grid, 1, _RB), Wfc,
      bfc.reshape(1, _C))

    return out
